# baseline (speedup 1.0000x reference)
import functools

import jax
import jax.numpy as jnp
from jax import lax
from jax.experimental import pallas as pl
from jax.experimental.pallas import tpu as pltpu
from jax.experimental.pallas import tpu_sc as plsc

DIM = 128
L = 200
LP = 208
EPS = 1e-12

NCORES = 2
NSUB = 16
NW = NCORES * NSUB
NCH = DIM // 16


GRP = 4
LPH = LP // 2


def _sc_pool(W, idx_flat, rows):
    rows_per_w = rows // NW
    mesh = plsc.VectorSubcoreMesh(core_axis_name="c", subcore_axis_name="s")

    @functools.partial(
        pl.kernel,
        out_type=jax.ShapeDtypeStruct((rows, DIM), jnp.float32),
        mesh=mesh,
        scratch_types=[
            pltpu.VMEM((rows_per_w * LP,), jnp.int32),
            pltpu.VMEM((LPH, DIM), jnp.float32),
            pltpu.VMEM((LPH, DIM), jnp.float32),
            pltpu.VMEM((LPH, DIM), jnp.float32),
            pltpu.VMEM((LPH, DIM), jnp.float32),
            pltpu.VMEM((GRP, DIM), jnp.float32),
            pltpu.SemaphoreType.DMA,
            pltpu.SemaphoreType.DMA,
            pltpu.SemaphoreType.DMA,
            pltpu.SemaphoreType.DMA,
        ],
    )
    def pool_kernel(w_hbm, idx_hbm, out_hbm, idx_all, buf_a, buf_b, buf_c,
                    buf_d, obuf, sem_a, sem_b, sem_c, sem_d):
        wid = lax.axis_index("c") * NSUB + lax.axis_index("s")
        base = wid * rows_per_w
        pltpu.sync_copy(
            idx_hbm.at[pl.ds(pl.multiple_of(base * LP, 8), rows_per_w * LP)],
            idx_all)

        def fire(rr, half, buf, sem):
            off = pl.multiple_of(rr * LP + half * LPH, 8)
            return pltpu.async_copy(
                w_hbm.at[idx_all.at[pl.ds(off, LPH)]], buf, sem)

        def wait(buf, sem):
            pltpu.make_async_copy(w_hbm.at[idx_all.at[pl.ds(0, LPH)]],
                                  buf, sem).wait()

        def accum(buf, acc):
            acc = tuple(acc[c] + buf[0, pl.ds(c * 16, 16)]
                        for c in range(NCH))

            def body(l, a):
                return tuple(a[c] + a[c] for c in range(NCH))

            return lax.fori_loop(0, LPH, body, acc)

        def consume(r, slot, b0, b1):
            acc = accum(
                b0, tuple(jnp.zeros((16,), jnp.float32) for _ in range(NCH)))
            acc = accum(b1, acc)
            for c in range(NCH):
                obuf[slot, pl.ds(c * 16, 16)] = acc[c]

        fire(0, 0, buf_a, sem_a)
        fire(0, 1, buf_b, sem_b)
        fire(1, 0, buf_c, sem_c)
        fire(1, 1, buf_d, sem_d)

        @pl.loop(0, rows_per_w, step=2)
        def _(r0):
            wait(buf_a, sem_a)
            wait(buf_b, sem_b)
            wait(buf_c, sem_c)
            wait(buf_d, sem_d)
            consume(r0, 0, buf_a, buf_b)
            consume(r0 + 1, 1, buf_c, buf_d)

            @pl.when(r0 + 2 < rows_per_w)
            def _():
                fire(r0 + 2, 0, buf_a, sem_a)
                fire(r0 + 2, 1, buf_b, sem_b)
                fire(r0 + 3, 0, buf_c, sem_c)
                fire(r0 + 3, 1, buf_d, sem_d)

            pltpu.sync_copy(obuf.at[pl.ds(0, 2)],
                            out_hbm.at[pl.ds(base + r0, 2)])

    return pool_kernel(W, idx_flat)


def _tc_norm(psum, idx, gamma, beta, rows):
    blk = 256

    def body(ps_ref, idx_ref, g_ref, b_ref, o_ref):
        s = ps_ref[...]
        cnt = jnp.sum((idx_ref[...] != 0).astype(jnp.float32), axis=1,
                      keepdims=True)
        p = s / cnt
        mu = jnp.mean(p, axis=1, keepdims=True)
        var = jnp.mean((p - mu) ** 2, axis=1, keepdims=True)
        o_ref[...] = (p - mu) * lax.rsqrt(var + EPS) * g_ref[...] + b_ref[...]

    return pl.pallas_call(
        body,
        grid=(rows // blk,),
        in_specs=[
            pl.BlockSpec((blk, DIM), lambda i: (i, 0)),
            pl.BlockSpec((blk, L), lambda i: (i, 0)),
            pl.BlockSpec((1, DIM), lambda i: (0, 0)),
            pl.BlockSpec((1, DIM), lambda i: (0, 0)),
        ],
        out_specs=pl.BlockSpec((blk, DIM), lambda i: (i, 0)),
        out_shape=jax.ShapeDtypeStruct((rows, DIM), jnp.float32),
    )(psum, idx, gamma.reshape(1, DIM), beta.reshape(1, DIM))


def kernel(x_s, x_t, W, gamma, beta):
    b = x_s.shape[0]
    rows = 2 * b
    idx = jnp.concatenate([x_s, x_t], axis=0)
    idx_flat = jnp.pad(idx, ((0, 0), (0, LP - L))).reshape(-1)
    psum = _sc_pool(W, idx_flat, rows)
    out = _tc_norm(psum, idx, gamma, beta, rows)
    return out[:b], out[b:]


# speedup vs baseline: 1.0007x; 1.0007x over previous
import functools

import jax
import jax.numpy as jnp
from jax import lax
from jax.experimental import pallas as pl
from jax.experimental.pallas import tpu as pltpu
from jax.experimental.pallas import tpu_sc as plsc

DIM = 128
L = 200
LP = 208
EPS = 1e-12

NCORES = 2
NSUB = 16
NW = NCORES * NSUB
NCH = DIM // 16


GRP = 4
LPH = LP // 2


def _sc_pool(W, idx_flat, rows):
    rows_per_w = rows // NW
    mesh = plsc.VectorSubcoreMesh(core_axis_name="c", subcore_axis_name="s")

    @functools.partial(
        pl.kernel,
        out_type=jax.ShapeDtypeStruct((rows, DIM), jnp.float32),
        mesh=mesh,
        scratch_types=[
            pltpu.VMEM((rows_per_w * LP,), jnp.int32),
            pltpu.VMEM((LPH, DIM), jnp.float32),
            pltpu.VMEM((LPH, DIM), jnp.float32),
            pltpu.VMEM((LPH, DIM), jnp.float32),
            pltpu.VMEM((LPH, DIM), jnp.float32),
            pltpu.VMEM((GRP, DIM), jnp.float32),
            pltpu.SemaphoreType.DMA,
            pltpu.SemaphoreType.DMA,
            pltpu.SemaphoreType.DMA,
            pltpu.SemaphoreType.DMA,
        ],
    )
    def pool_kernel(w_hbm, idx_hbm, out_hbm, idx_all, buf_a, buf_b, buf_c,
                    buf_d, obuf, sem_a, sem_b, sem_c, sem_d):
        wid = lax.axis_index("c") * NSUB + lax.axis_index("s")
        base = wid * rows_per_w
        pltpu.sync_copy(
            idx_hbm.at[pl.ds(pl.multiple_of(base * LP, 8), rows_per_w * LP)],
            idx_all)

        def fire(rr, half, buf, sem):
            off = pl.multiple_of(rr * LP + half * LPH, 8)
            return pltpu.async_copy(
                w_hbm.at[idx_all.at[pl.ds(off, LPH)]], buf, sem)

        def wait(buf, sem):
            pltpu.make_async_copy(w_hbm.at[idx_all.at[pl.ds(0, LPH)]],
                                  buf, sem).wait()

        def accum(buf, acc):
            def body(l, a):
                return tuple(
                    a[c] + buf[l, pl.ds(c * 16, 16)] for c in range(NCH)
                )

            return lax.fori_loop(0, LPH, body, acc)

        def consume(r, slot, b0, b1):
            acc = accum(
                b0, tuple(jnp.zeros((16,), jnp.float32) for _ in range(NCH)))
            acc = accum(b1, acc)
            for c in range(NCH):
                obuf[slot, pl.ds(c * 16, 16)] = acc[c]

        fire(0, 0, buf_a, sem_a)
        fire(0, 1, buf_b, sem_b)
        fire(1, 0, buf_c, sem_c)
        fire(1, 1, buf_d, sem_d)

        @pl.loop(0, rows_per_w, step=2)
        def _(r0):
            with jax.named_scope("wait4"):
                wait(buf_a, sem_a)
                wait(buf_b, sem_b)
                wait(buf_c, sem_c)
                wait(buf_d, sem_d)
            with jax.named_scope("accum2"):
                consume(r0, 0, buf_a, buf_b)
                consume(r0 + 1, 1, buf_c, buf_d)

            @pl.when(r0 + 2 < rows_per_w)
            def _():
                with jax.named_scope("fire4"):
                    fire(r0 + 2, 0, buf_a, sem_a)
                    fire(r0 + 2, 1, buf_b, sem_b)
                    fire(r0 + 3, 0, buf_c, sem_c)
                    fire(r0 + 3, 1, buf_d, sem_d)

            with jax.named_scope("flush"):
                pltpu.sync_copy(obuf.at[pl.ds(0, 2)],
                                out_hbm.at[pl.ds(base + r0, 2)])

    return pool_kernel(W, idx_flat)


def _tc_norm(psum, idx, gamma, beta, rows):
    blk = 256

    def body(ps_ref, idx_ref, g_ref, b_ref, o_ref):
        s = ps_ref[...]
        cnt = jnp.sum((idx_ref[...] != 0).astype(jnp.float32), axis=1,
                      keepdims=True)
        p = s / cnt
        mu = jnp.mean(p, axis=1, keepdims=True)
        var = jnp.mean((p - mu) ** 2, axis=1, keepdims=True)
        o_ref[...] = (p - mu) * lax.rsqrt(var + EPS) * g_ref[...] + b_ref[...]

    return pl.pallas_call(
        body,
        grid=(rows // blk,),
        in_specs=[
            pl.BlockSpec((blk, DIM), lambda i: (i, 0)),
            pl.BlockSpec((blk, L), lambda i: (i, 0)),
            pl.BlockSpec((1, DIM), lambda i: (0, 0)),
            pl.BlockSpec((1, DIM), lambda i: (0, 0)),
        ],
        out_specs=pl.BlockSpec((blk, DIM), lambda i: (i, 0)),
        out_shape=jax.ShapeDtypeStruct((rows, DIM), jnp.float32),
    )(psum, idx, gamma.reshape(1, DIM), beta.reshape(1, DIM))


def kernel(x_s, x_t, W, gamma, beta):
    b = x_s.shape[0]
    rows = 2 * b
    idx = jnp.concatenate([x_s, x_t], axis=0)
    idx_flat = jnp.pad(idx, ((0, 0), (0, LP - L))).reshape(-1)
    psum = _sc_pool(W, idx_flat, rows)
    out = _tc_norm(psum, idx, gamma, beta, rows)
    return out[:b], out[b:]


# speedup vs baseline: 1.0065x; 1.0058x over previous
import functools

import jax
import jax.numpy as jnp
from jax import lax
from jax.experimental import pallas as pl
from jax.experimental.pallas import tpu as pltpu
from jax.experimental.pallas import tpu_sc as plsc

DIM = 128
L = 200
LP = 208
EPS = 1e-12

NCORES = 2
NSUB = 16
NW = NCORES * NSUB
NCH = DIM // 16


GRP = 4
LPH = LP // 2


def _sc_pool(W, idx_flat, rows):
    rows_per_w = rows // NW
    mesh = plsc.VectorSubcoreMesh(core_axis_name="c", subcore_axis_name="s")

    @functools.partial(
        pl.kernel,
        out_type=jax.ShapeDtypeStruct((rows, DIM), jnp.float32),
        mesh=mesh,
        scratch_types=[
            pltpu.VMEM((rows_per_w * LP,), jnp.int32),
            pltpu.VMEM((LPH, DIM), jnp.float32),
            pltpu.VMEM((LPH, DIM), jnp.float32),
            pltpu.VMEM((LPH, DIM), jnp.float32),
            pltpu.VMEM((LPH, DIM), jnp.float32),
            pltpu.VMEM((GRP, DIM), jnp.float32),
            pltpu.SemaphoreType.DMA,
            pltpu.SemaphoreType.DMA,
            pltpu.SemaphoreType.DMA,
            pltpu.SemaphoreType.DMA,
        ],
    )
    def pool_kernel(w_hbm, idx_hbm, out_hbm, idx_all, buf_a, buf_b, buf_c,
                    buf_d, obuf, sem_a, sem_b, sem_c, sem_d):
        wid = lax.axis_index("c") * NSUB + lax.axis_index("s")
        base = wid * rows_per_w
        pltpu.sync_copy(
            idx_hbm.at[pl.ds(pl.multiple_of(base * LP, 8), rows_per_w * LP)],
            idx_all)

        def fire(rr, half, buf, sem):
            off = pl.multiple_of(rr * LP + half * LPH, 8)
            return pltpu.async_copy(
                w_hbm.at[idx_all.at[pl.ds(off, LPH)]], buf, sem)

        def wait(buf, sem):
            pltpu.make_async_copy(w_hbm.at[idx_all.at[pl.ds(0, LPH)]],
                                  buf, sem).wait()

        def accum(buf, acc):
            def body(l, a):
                return tuple(
                    a[c] + buf[l, pl.ds(c * 16, 16)] for c in range(NCH)
                )

            return lax.fori_loop(0, LPH, body, acc)

        def consume(r, slot, b0, b1):
            acc = accum(
                b0, tuple(jnp.zeros((16,), jnp.float32) for _ in range(NCH)))
            acc = accum(b1, acc)
            for c in range(NCH):
                obuf[slot, pl.ds(c * 16, 16)] = acc[c]

        fire(0, 0, buf_a, sem_a)
        fire(0, 1, buf_b, sem_b)

        @pl.loop(0, rows_per_w, step=2)
        def _(r0):
            with jax.named_scope("wait4"):
                wait(buf_a, sem_a)
                wait(buf_b, sem_b)
            with jax.named_scope("accum2"):
                consume(r0, 0, buf_a, buf_b)

            @pl.when(r0 + 1 < rows_per_w)
            def _():
                with jax.named_scope("fire4"):
                    fire(r0 + 1, 0, buf_c, sem_c)
                    fire(r0 + 1, 1, buf_d, sem_d)

            with jax.named_scope("wait4b"):
                wait(buf_c, sem_c)
                wait(buf_d, sem_d)
            with jax.named_scope("accum2b"):
                consume(r0 + 1, 1, buf_c, buf_d)

            @pl.when(r0 + 2 < rows_per_w)
            def _():
                with jax.named_scope("fire4b"):
                    fire(r0 + 2, 0, buf_a, sem_a)
                    fire(r0 + 2, 1, buf_b, sem_b)

            with jax.named_scope("flush"):
                pltpu.sync_copy(obuf.at[pl.ds(0, 2)],
                                out_hbm.at[pl.ds(base + r0, 2)])

    return pool_kernel(W, idx_flat)


def _tc_norm(psum, idx, gamma, beta, rows):
    blk = 256

    def body(ps_ref, idx_ref, g_ref, b_ref, o_ref):
        s = ps_ref[...]
        cnt = jnp.sum((idx_ref[...] != 0).astype(jnp.float32), axis=1,
                      keepdims=True)
        p = s / cnt
        mu = jnp.mean(p, axis=1, keepdims=True)
        var = jnp.mean((p - mu) ** 2, axis=1, keepdims=True)
        o_ref[...] = (p - mu) * lax.rsqrt(var + EPS) * g_ref[...] + b_ref[...]

    return pl.pallas_call(
        body,
        grid=(rows // blk,),
        in_specs=[
            pl.BlockSpec((blk, DIM), lambda i: (i, 0)),
            pl.BlockSpec((blk, L), lambda i: (i, 0)),
            pl.BlockSpec((1, DIM), lambda i: (0, 0)),
            pl.BlockSpec((1, DIM), lambda i: (0, 0)),
        ],
        out_specs=pl.BlockSpec((blk, DIM), lambda i: (i, 0)),
        out_shape=jax.ShapeDtypeStruct((rows, DIM), jnp.float32),
    )(psum, idx, gamma.reshape(1, DIM), beta.reshape(1, DIM))


def kernel(x_s, x_t, W, gamma, beta):
    b = x_s.shape[0]
    rows = 2 * b
    idx = jnp.concatenate([x_s, x_t], axis=0)
    idx_flat = jnp.pad(idx, ((0, 0), (0, LP - L))).reshape(-1)
    psum = _sc_pool(W, idx_flat, rows)
    out = _tc_norm(psum, idx, gamma, beta, rows)
    return out[:b], out[b:]


# speedup vs baseline: 1.0069x; 1.0004x over previous
import functools

import jax
import jax.numpy as jnp
from jax import lax
from jax.experimental import pallas as pl
from jax.experimental.pallas import tpu as pltpu
from jax.experimental.pallas import tpu_sc as plsc

DIM = 128
L = 200
LP = 208
EPS = 1e-12

NCORES = 2
NSUB = 16
NW = NCORES * NSUB
NCH = DIM // 16


GRP = 4
LPH = LP // 2


def _sc_pool(W, idx_flat, rows):
    rows_per_w = rows // NW
    mesh = plsc.VectorSubcoreMesh(core_axis_name="c", subcore_axis_name="s")

    @functools.partial(
        pl.kernel,
        out_type=jax.ShapeDtypeStruct((rows, DIM), jnp.float32),
        mesh=mesh,
        scratch_types=[
            pltpu.VMEM((rows_per_w * LP,), jnp.int32),
            pltpu.VMEM((LPH, DIM), jnp.float32),
            pltpu.VMEM((LPH, DIM), jnp.float32),
            pltpu.VMEM((LPH, DIM), jnp.float32),
            pltpu.VMEM((LPH, DIM), jnp.float32),
            pltpu.VMEM((GRP, DIM), jnp.float32),
            pltpu.SemaphoreType.DMA,
            pltpu.SemaphoreType.DMA,
            pltpu.SemaphoreType.DMA,
            pltpu.SemaphoreType.DMA,
        ],
    )
    def pool_kernel(w_hbm, idx_hbm, out_hbm, idx_all, buf_a, buf_b, buf_c,
                    buf_d, obuf, sem_a, sem_b, sem_c, sem_d):
        wid = lax.axis_index("c") * NSUB + lax.axis_index("s")
        base = wid * rows_per_w
        pltpu.sync_copy(
            idx_hbm.at[pl.ds(pl.multiple_of(base * LP, 8), rows_per_w * LP)],
            idx_all)

        def fire(rr, half, buf, sem):
            off = pl.multiple_of(rr * LP + half * LPH, 8)
            return pltpu.async_copy(
                w_hbm.at[idx_all.at[pl.ds(off, LPH)]], buf, sem)

        def wait(buf, sem):
            pltpu.make_async_copy(w_hbm.at[idx_all.at[pl.ds(0, LPH)]],
                                  buf, sem).wait()

        def accum(buf, acc):
            def body(l, a):
                return tuple(
                    a[c] + buf[l, pl.ds(c * 16, 16)] for c in range(NCH)
                )

            return lax.fori_loop(0, LPH, body, acc)

        def consume(r, slot, b0, b1):
            acc = accum(
                b0, tuple(jnp.zeros((16,), jnp.float32) for _ in range(NCH)))
            acc = accum(b1, acc)
            for c in range(NCH):
                obuf[slot, pl.ds(c * 16, 16)] = acc[c]

        fire(0, 0, buf_a, sem_a)
        fire(0, 1, buf_b, sem_b)

        fire(1, 0, buf_c, sem_c)
        fire(1, 1, buf_d, sem_d)

        @pl.loop(0, rows_per_w, step=2)
        def _(r0):
            with jax.named_scope("wait4"):
                wait(buf_a, sem_a)
                wait(buf_b, sem_b)
                wait(buf_c, sem_c)
                wait(buf_d, sem_d)
            plsc.subcore_barrier()
            with jax.named_scope("accum2"):
                consume(r0, 0, buf_a, buf_b)
                consume(r0 + 1, 1, buf_c, buf_d)
            plsc.subcore_barrier()

            @pl.when(r0 + 2 < rows_per_w)
            def _():
                with jax.named_scope("fire4"):
                    fire(r0 + 2, 0, buf_a, sem_a)
                    fire(r0 + 2, 1, buf_b, sem_b)
                    fire(r0 + 3, 0, buf_c, sem_c)
                    fire(r0 + 3, 1, buf_d, sem_d)

            with jax.named_scope("flush"):
                pltpu.sync_copy(obuf.at[pl.ds(0, 2)],
                                out_hbm.at[pl.ds(base + r0, 2)])

    return pool_kernel(W, idx_flat)


def _tc_norm(psum, idx, gamma, beta, rows):
    blk = 256

    def body(ps_ref, idx_ref, g_ref, b_ref, o_ref):
        s = ps_ref[...]
        cnt = jnp.sum((idx_ref[...] != 0).astype(jnp.float32), axis=1,
                      keepdims=True)
        p = s / cnt
        mu = jnp.mean(p, axis=1, keepdims=True)
        var = jnp.mean((p - mu) ** 2, axis=1, keepdims=True)
        o_ref[...] = (p - mu) * lax.rsqrt(var + EPS) * g_ref[...] + b_ref[...]

    return pl.pallas_call(
        body,
        grid=(rows // blk,),
        in_specs=[
            pl.BlockSpec((blk, DIM), lambda i: (i, 0)),
            pl.BlockSpec((blk, L), lambda i: (i, 0)),
            pl.BlockSpec((1, DIM), lambda i: (0, 0)),
            pl.BlockSpec((1, DIM), lambda i: (0, 0)),
        ],
        out_specs=pl.BlockSpec((blk, DIM), lambda i: (i, 0)),
        out_shape=jax.ShapeDtypeStruct((rows, DIM), jnp.float32),
    )(psum, idx, gamma.reshape(1, DIM), beta.reshape(1, DIM))


def kernel(x_s, x_t, W, gamma, beta):
    b = x_s.shape[0]
    rows = 2 * b
    idx = jnp.concatenate([x_s, x_t], axis=0)
    idx_flat = jnp.pad(idx, ((0, 0), (0, LP - L))).reshape(-1)
    psum = _sc_pool(W, idx_flat, rows)
    out = _tc_norm(psum, idx, gamma, beta, rows)
    return out[:b], out[b:]
